# baseline (speedup 1.0000x reference)
import numpy as np

import jax
import jax.numpy as jnp
from jax.experimental import pallas as pl
from jax.experimental.pallas import tpu as pltpu

_F32 = jnp.float32


def _cdiv(a, b):
    return -(-a // b)


def _placement(kw, w_in, pw, parity):
    j = np.arange(kw)[:, None, None]
    x = np.arange(w_in)[None, :, None]
    c = np.arange(pw)[None, None, :]
    return (x == 2 * c + parity + j).astype(np.float32)


_R1 = [_placement(5, 32, 14, p) for p in (0, 1)]
_R2 = [_placement(5, 14, 5, p) for p in (0, 1)]

_T1 = (np.arange(8)[None, :, None]
       == np.arange(4)[:, None, None] + np.arange(5)[None, None, :]
       ).astype(np.float32)

_T2 = (2 * np.arange(3)[None, :, None, None] + np.arange(2)[None, None, :, None]
       - np.arange(2)[:, None, None, None] == np.arange(5)[None, None, None, :]
       ).astype(np.float32)


_BF16 = jnp.bfloat16


def _fused_kernel(x_ref, w1_ref, c1b_ref, w2_ref, c2b_ref, w3_ref, b3_ref,
                  w4_ref, b4_ref, w5_ref, b5_ref, o_ref):
    tb = x_ref.shape[0]
    xb = x_ref[...].astype(_BF16)

    acc = None
    for i in range(5):
        rolled = xb if i == 0 else jnp.roll(xb, -i, axis=1)
        part = jnp.dot(rolled.reshape(tb * 32, 32),
                       w1_ref[i * 32:(i + 1) * 32, :],
                       preferred_element_type=_F32)
        acc = part if acc is None else acc + part
    acc = acc.reshape(tb, 32, 256)
    acc = jnp.maximum(acc[:, :, 0:128], acc[:, :, 128:256])
    acc = acc.reshape(tb, 16, 2, 128).max(axis=2)
    a1 = jnp.maximum(acc + c1b_ref[...], 0.0).astype(_BF16)

    acc = None
    for i in range(5):
        rolled = a1 if i == 0 else jnp.roll(a1, -i, axis=1)
        part = jnp.dot(rolled.reshape(tb * 16, 128),
                       w2_ref[i * 128:(i + 1) * 128, :],
                       preferred_element_type=_F32)
        acc = part if acc is None else acc + part
    acc = acc.reshape(tb, 16, 256)
    acc = jnp.maximum(acc[:, :, 0:128], acc[:, :, 128:256])
    acc = acc.reshape(tb, 8, 2, 128).max(axis=2)
    a2 = jnp.maximum(acc + c2b_ref[...], 0.0).astype(_BF16)

    h = None
    for i in range(5):
        part = jnp.dot(a2[:, i, :], w3_ref[i * 128:(i + 1) * 128, :],
                       preferred_element_type=_F32)
        h = part if h is None else h + part
    h = jnp.maximum(h + b3_ref[...], 0.0).astype(_BF16)
    h = jnp.dot(h, w4_ref[...], preferred_element_type=_F32)
    h = jnp.maximum(h + b4_ref[...], 0.0).astype(_BF16)
    h = jnp.dot(h, w5_ref[...], preferred_element_type=_F32)
    o_ref[...] = (h + b5_ref[...]).astype(o_ref.dtype)


def kernel(x, w1, b1, w2, b2, w3, b3, w4, b4, w5, b5):
    B = x.shape[0]
    xs = x.reshape(B, 32, 32)

    w1t = jnp.transpose(w1.reshape(6, 5, 5), (1, 2, 0)).astype(_F32)
    w1_par = []
    for p in (0, 1):
        m = jnp.einsum('jxc,ijo->ixco', jnp.asarray(_R1[p]), w1t)
        w1_par.append(jnp.pad(m.reshape(5, 32, 84), ((0, 0), (0, 0), (0, 44))))
    w1m = jnp.concatenate(w1_par, axis=-1).reshape(160, 256).astype(_BF16)

    w2t = jnp.transpose(w2, (2, 3, 1, 0)).astype(_F32)
    w2_par = []
    for p in (0, 1):
        m = jnp.einsum('jrc,ijao->iraco', jnp.asarray(_R2[p]), w2t)
        w2_par.append(jnp.pad(m.reshape(5, 84, 80), ((0, 0), (0, 44), (0, 48))))
    w2m = jnp.concatenate(w2_par, axis=-1).reshape(640, 256).astype(_BF16)

    c1b = jnp.pad(jnp.tile(b1.astype(_F32), 14), (0, 44)).reshape(1, 128)
    c2b = jnp.pad(jnp.tile(b2.astype(_F32), 5), (0, 48)).reshape(1, 128)

    w3t = jnp.transpose(w3, (2, 3, 1, 0)).reshape(5, 80, 120).astype(_F32)
    w3m = jnp.pad(w3t, ((0, 0), (0, 48), (0, 0))).reshape(640, 120).astype(_BF16)
    w4t = w4.T.astype(_BF16)
    w5t = w5.T.astype(_BF16)
    b3r = b3.reshape(1, 120).astype(_F32)
    b4r = b4.reshape(1, 84).astype(_F32)
    b5r = b5.reshape(1, 10).astype(_F32)

    tb = 512
    nb = _cdiv(B, tb)
    b_pad = nb * tb
    if b_pad != B:
        xs = jnp.pad(xs, ((0, b_pad - B), (0, 0), (0, 0)))

    out = pl.pallas_call(
        _fused_kernel,
        out_shape=jax.ShapeDtypeStruct((b_pad, 10), _F32),
        grid_spec=pltpu.PrefetchScalarGridSpec(
            num_scalar_prefetch=0,
            grid=(nb,),
            in_specs=[
                pl.BlockSpec((tb, 32, 32), lambda m: (m, 0, 0)),
                pl.BlockSpec((160, 256), lambda m: (0, 0)),
                pl.BlockSpec((1, 128), lambda m: (0, 0)),
                pl.BlockSpec((640, 256), lambda m: (0, 0)),
                pl.BlockSpec((1, 128), lambda m: (0, 0)),
                pl.BlockSpec((640, 120), lambda m: (0, 0)),
                pl.BlockSpec((1, 120), lambda m: (0, 0)),
                pl.BlockSpec((120, 84), lambda m: (0, 0)),
                pl.BlockSpec((1, 84), lambda m: (0, 0)),
                pl.BlockSpec((84, 10), lambda m: (0, 0)),
                pl.BlockSpec((1, 10), lambda m: (0, 0)),
            ],
            out_specs=pl.BlockSpec((tb, 10), lambda m: (m, 0)),
        ),
        compiler_params=pltpu.CompilerParams(
            dimension_semantics=("parallel",),
            vmem_limit_bytes=64 * 1024 * 1024,
        ),
        cost_estimate=pl.CostEstimate(
            flops=2 * b_pad * (28 * 160 * 256 + 10 * 640 * 256 + 640 * 120
                               + 120 * 84 + 84 * 10),
            transcendentals=0,
            bytes_accessed=4 * (b_pad * 32 * 32 + b_pad * 10),
        ),
    )(xs, w1m, c1b, w2m, c2b, w3m, b3r, w4t, b4r, w5t, b5r)
    return out[:B]


# speedup vs baseline: 3.7486x; 3.7486x over previous
import numpy as np

import jax
import jax.numpy as jnp
from jax.experimental import pallas as pl
from jax.experimental.pallas import tpu as pltpu

_F32 = jnp.float32
_BF16 = jnp.bfloat16


def _cdiv(a, b):
    return -(-a // b)


def _placement(kw, w_in, pw, parity):
    j = np.arange(kw)[:, None, None]
    x = np.arange(w_in)[None, :, None]
    c = np.arange(pw)[None, None, :]
    return (x == 2 * c + parity + j).astype(np.float32)


_R1 = [_placement(5, 32, 14, p) for p in (0, 1)]
_R2 = [_placement(5, 14, 5, p) for p in (0, 1)]

_T1 = (np.arange(8)[None, :, None]
       == np.arange(4)[:, None, None] + np.arange(5)[None, None, :]
       ).astype(np.float32)

_T2 = (2 * np.arange(3)[None, :, None, None] + np.arange(2)[None, None, :, None]
       - np.arange(2)[:, None, None, None] == np.arange(5)[None, None, None, :]
       ).astype(np.float32)


def _fused_kernel(x_ref, w1_ref, c1b_ref, w2_ref, c2b_ref, w3_ref, b3_ref,
                  w4_ref, b4_ref, w5_ref, b5_ref, o_ref):
    tb = x_ref.shape[0] // 4

    x4 = x_ref[...].astype(_BF16).reshape(tb, 4, 8, 32)
    parts = []
    for r4 in range(4):
        parts.append(jnp.concatenate(
            [x4[:, :, r4:r4 + 1, :], x4[:, :, r4 + 4:r4 + 5, :]],
            axis=2).reshape(tb, 8, 32))
    x8 = jnp.concatenate(parts, axis=-1)

    lhs = jnp.concatenate([x8, jnp.roll(x8, -1, axis=1)], axis=-1)
    acc = jnp.dot(lhs.reshape(tb * 8, 256), w1_ref[...],
                  preferred_element_type=_F32).reshape(tb, 8, 1024)
    acc = jnp.maximum(acc[:, :, 0:512], acc[:, :, 512:1024])
    acc = jnp.maximum(acc[:, :, 0:256], acc[:, :, 256:512])
    a1 = jnp.maximum(acc + c1b_ref[...], 0.0).astype(_BF16)

    lhs = jnp.concatenate(
        [a1, jnp.roll(a1, -1, axis=1), jnp.roll(a1, -2, axis=1)], axis=-1)
    acc = jnp.dot(lhs.reshape(tb * 8, 768), w2_ref[...],
                  preferred_element_type=_F32).reshape(tb, 8, 512)
    acc = jnp.maximum(acc[:, :, 0:256], acc[:, :, 256:512])
    acc = jnp.maximum(acc[:, :, 0:128], acc[:, :, 128:256])
    a2 = jnp.maximum(acc + c2b_ref[...], 0.0).astype(_BF16)

    h = None
    for i in range(5):
        part = jnp.dot(a2[:, i, :], w3_ref[i * 128:(i + 1) * 128, :],
                       preferred_element_type=_F32)
        h = part if h is None else h + part
    h = jnp.maximum(h + b3_ref[...], 0.0).astype(_BF16)
    h = jnp.dot(h, w4_ref[...], preferred_element_type=_F32)
    h = jnp.maximum(h + b4_ref[...], 0.0).astype(_BF16)
    h = jnp.dot(h, w5_ref[...], preferred_element_type=_F32)
    o_ref[...] = (h + b5_ref[...]).astype(o_ref.dtype)


def kernel(x, w1, b1, w2, b2, w3, b3, w4, b4, w5, b5):
    B = x.shape[0]
    xs = x.reshape(B * 4, 8, 32)

    w1t = jnp.transpose(w1.reshape(6, 5, 5), (1, 2, 0)).astype(_F32)
    w1_par = []
    for p in (0, 1):
        m = jnp.einsum('qdi,jwc,ijo->dwqco', jnp.asarray(_T1),
                       jnp.asarray(_R1[p]), w1t)
        m = m.reshape(8, 32, 2, 2, 84).transpose(0, 1, 3, 2, 4)
        w1_par.append(jnp.pad(m, ((0, 0),) * 4 + ((0, 44),)))
    w1m = jnp.stack(w1_par, axis=2).reshape(256, 1024).astype(_BF16)

    w2t = jnp.transpose(w2, (2, 3, 1, 0)).astype(_F32)
    w2_par = []
    for p in (0, 1):
        m = jnp.einsum('qgai,jrc,ijno->garnqco', jnp.asarray(_T2),
                       jnp.asarray(_R2[p]), w2t)
        w2_par.append(jnp.pad(m.reshape(3, 2, 84, 2, 80),
                              ((0, 0), (0, 0), (0, 44), (0, 0), (0, 48))))
    w2m = jnp.stack(w2_par, axis=3).reshape(768, 512).astype(_BF16)

    c1b = jnp.tile(jnp.pad(jnp.tile(b1.astype(_F32), 14), (0, 44)),
                   2).reshape(1, 256)
    c2b = jnp.pad(jnp.tile(b2.astype(_F32), 5), (0, 48)).reshape(1, 128)

    w3t = jnp.transpose(w3, (2, 3, 1, 0)).reshape(5, 80, 120).astype(_F32)
    w3m = jnp.pad(w3t, ((0, 0), (0, 48), (0, 0))).reshape(640, 120).astype(_BF16)
    w4t = w4.T.astype(_BF16)
    w5t = w5.T.astype(_BF16)
    b3r = b3.reshape(1, 120).astype(_F32)
    b4r = b4.reshape(1, 84).astype(_F32)
    b5r = b5.reshape(1, 10).astype(_F32)

    tb = 512
    nb = _cdiv(B, tb)
    b_pad = nb * tb
    if b_pad != B:
        xs = jnp.pad(xs, ((0, 4 * (b_pad - B)), (0, 0), (0, 0)))

    out = pl.pallas_call(
        _fused_kernel,
        out_shape=jax.ShapeDtypeStruct((b_pad, 10), _F32),
        grid_spec=pltpu.PrefetchScalarGridSpec(
            num_scalar_prefetch=0,
            grid=(nb,),
            in_specs=[
                pl.BlockSpec((tb * 4, 8, 32), lambda m: (m, 0, 0)),
                pl.BlockSpec((256, 1024), lambda m: (0, 0)),
                pl.BlockSpec((1, 256), lambda m: (0, 0)),
                pl.BlockSpec((768, 512), lambda m: (0, 0)),
                pl.BlockSpec((1, 128), lambda m: (0, 0)),
                pl.BlockSpec((640, 120), lambda m: (0, 0)),
                pl.BlockSpec((1, 120), lambda m: (0, 0)),
                pl.BlockSpec((120, 84), lambda m: (0, 0)),
                pl.BlockSpec((1, 84), lambda m: (0, 0)),
                pl.BlockSpec((84, 10), lambda m: (0, 0)),
                pl.BlockSpec((1, 10), lambda m: (0, 0)),
            ],
            out_specs=pl.BlockSpec((tb, 10), lambda m: (m, 0)),
        ),
        compiler_params=pltpu.CompilerParams(
            dimension_semantics=("parallel",),
            vmem_limit_bytes=64 * 1024 * 1024,
        ),
        cost_estimate=pl.CostEstimate(
            flops=2 * b_pad * (8 * 256 * 1024 + 8 * 768 * 512
                               + 5 * 128 * 128 + 128 * 128 + 128 * 128),
            transcendentals=0,
            bytes_accessed=4 * (b_pad * 32 * 32 + b_pad * 10),
        ),
    )(xs, w1m, c1b, w2m, c2b, w3m, b3r, w4t, b4r, w5t, b5r)
    return out[:B]


# speedup vs baseline: 5.0202x; 1.3392x over previous
import numpy as np

import jax
import jax.numpy as jnp
from jax.experimental import pallas as pl
from jax.experimental.pallas import tpu as pltpu

_F32 = jnp.float32
_BF16 = jnp.bfloat16


def _cdiv(a, b):
    return -(-a // b)


def _placement(kw, w_in, pw, parity):
    j = np.arange(kw)[:, None, None]
    x = np.arange(w_in)[None, :, None]
    c = np.arange(pw)[None, None, :]
    return (x == 2 * c + parity + j).astype(np.float32)


_R1 = [_placement(5, 32, 14, p) for p in (0, 1)]
_R2 = [_placement(5, 14, 5, p) for p in (0, 1)]

_T1 = (np.arange(8)[None, :, None]
       == np.arange(4)[:, None, None] + np.arange(5)[None, None, :]
       ).astype(np.float32)

_T2 = (2 * np.arange(3)[None, :, None, None] + np.arange(2)[None, None, :, None]
       - np.arange(2)[:, None, None, None] == np.arange(5)[None, None, None, :]
       ).astype(np.float32)


def _fused_kernel(x_ref, w1_ref, c1b_ref, w2_ref, c2b_ref, w3_ref, b3_ref,
                  w4_ref, b4_ref, w5_ref, b5_ref, o_ref):
    tb = x_ref.shape[0]
    x8 = x_ref[...]

    lhs = jnp.concatenate([x8, jnp.roll(x8, -1, axis=1)], axis=-1)
    acc = jnp.dot(lhs.reshape(tb * 8, 256), w1_ref[...],
                  preferred_element_type=_F32).reshape(tb, 8, 1024)
    acc = jnp.maximum(acc[:, :, 0:512], acc[:, :, 512:1024])
    acc = jnp.maximum(acc[:, :, 0:256], acc[:, :, 256:512])
    a1 = jnp.maximum(acc + c1b_ref[...], 0.0).astype(_BF16)

    lhs = jnp.concatenate(
        [a1, jnp.roll(a1, -1, axis=1), jnp.roll(a1, -2, axis=1)], axis=-1)
    acc = jnp.dot(lhs.reshape(tb * 8, 768), w2_ref[...],
                  preferred_element_type=_F32).reshape(tb, 8, 512)
    acc = jnp.maximum(acc[:, :, 0:256], acc[:, :, 256:512])
    acc = jnp.maximum(acc[:, :, 0:128], acc[:, :, 128:256])
    a2 = jnp.maximum(acc + c2b_ref[...], 0.0).astype(_BF16)

    h = None
    for i in range(5):
        part = jnp.dot(a2[:, i, :], w3_ref[i * 128:(i + 1) * 128, :],
                       preferred_element_type=_F32)
        h = part if h is None else h + part
    h = jnp.maximum(h + b3_ref[...], 0.0).astype(_BF16)
    h = jnp.dot(h, w4_ref[...], preferred_element_type=_F32)
    h = jnp.maximum(h + b4_ref[...], 0.0).astype(_BF16)
    h = jnp.dot(h, w5_ref[...], preferred_element_type=_F32)
    o_ref[...] = (h + b5_ref[...]).astype(o_ref.dtype)


def kernel(x, w1, b1, w2, b2, w3, b3, w4, b4, w5, b5):
    B = x.shape[0]
    xs = x.reshape(B, 8, 128).astype(_BF16)

    w1t = jnp.transpose(w1.reshape(6, 5, 5), (1, 2, 0)).astype(_F32)
    w1_par = []
    for p in (0, 1):
        m = jnp.einsum('qdi,jwc,ijo->dwqco', jnp.asarray(_T1),
                       jnp.asarray(_R1[p]), w1t)
        m = m.reshape(8, 32, 2, 2, 84).transpose(0, 1, 3, 2, 4)
        w1_par.append(jnp.pad(m, ((0, 0),) * 4 + ((0, 44),)))
    w1m = jnp.stack(w1_par, axis=2).reshape(256, 1024).astype(_BF16)

    w2t = jnp.transpose(w2, (2, 3, 1, 0)).astype(_F32)
    w2_par = []
    for p in (0, 1):
        m = jnp.einsum('qgai,jrc,ijno->garnqco', jnp.asarray(_T2),
                       jnp.asarray(_R2[p]), w2t)
        w2_par.append(jnp.pad(m.reshape(3, 2, 84, 2, 80),
                              ((0, 0), (0, 0), (0, 44), (0, 0), (0, 48))))
    w2m = jnp.stack(w2_par, axis=3).reshape(768, 512).astype(_BF16)

    c1b = jnp.tile(jnp.pad(jnp.tile(b1.astype(_F32), 14), (0, 44)),
                   2).reshape(1, 256)
    c2b = jnp.pad(jnp.tile(b2.astype(_F32), 5), (0, 48)).reshape(1, 128)

    w3t = jnp.transpose(w3, (2, 3, 1, 0)).reshape(5, 80, 120).astype(_F32)
    w3m = jnp.pad(w3t, ((0, 0), (0, 48), (0, 0))).reshape(640, 120).astype(_BF16)
    w4t = w4.T.astype(_BF16)
    w5t = w5.T.astype(_BF16)
    b3r = b3.reshape(1, 120).astype(_F32)
    b4r = b4.reshape(1, 84).astype(_F32)
    b5r = b5.reshape(1, 10).astype(_F32)

    tb = 512
    nb = _cdiv(B, tb)
    b_pad = nb * tb
    if b_pad != B:
        xs = jnp.pad(xs, ((0, b_pad - B), (0, 0), (0, 0)))

    out = pl.pallas_call(
        _fused_kernel,
        out_shape=jax.ShapeDtypeStruct((b_pad, 10), _F32),
        grid_spec=pltpu.PrefetchScalarGridSpec(
            num_scalar_prefetch=0,
            grid=(nb,),
            in_specs=[
                pl.BlockSpec((tb, 8, 128), lambda m: (m, 0, 0)),
                pl.BlockSpec((256, 1024), lambda m: (0, 0)),
                pl.BlockSpec((1, 256), lambda m: (0, 0)),
                pl.BlockSpec((768, 512), lambda m: (0, 0)),
                pl.BlockSpec((1, 128), lambda m: (0, 0)),
                pl.BlockSpec((640, 120), lambda m: (0, 0)),
                pl.BlockSpec((1, 120), lambda m: (0, 0)),
                pl.BlockSpec((120, 84), lambda m: (0, 0)),
                pl.BlockSpec((1, 84), lambda m: (0, 0)),
                pl.BlockSpec((84, 10), lambda m: (0, 0)),
                pl.BlockSpec((1, 10), lambda m: (0, 0)),
            ],
            out_specs=pl.BlockSpec((tb, 10), lambda m: (m, 0)),
        ),
        compiler_params=pltpu.CompilerParams(
            dimension_semantics=("parallel",),
            vmem_limit_bytes=64 * 1024 * 1024,
        ),
        cost_estimate=pl.CostEstimate(
            flops=2 * b_pad * (8 * 256 * 1024 + 8 * 768 * 512
                               + 5 * 128 * 128 + 128 * 128 + 128 * 128),
            transcendentals=0,
            bytes_accessed=4 * (b_pad * 32 * 32 + b_pad * 10),
        ),
    )(xs, w1m, c1b, w2m, c2b, w3m, b3r, w4t, b4r, w5t, b5r)
    return out[:B]


# speedup vs baseline: 5.1386x; 1.0236x over previous
import numpy as np

import jax
import jax.numpy as jnp
from jax.experimental import pallas as pl
from jax.experimental.pallas import tpu as pltpu

_F32 = jnp.float32
_BF16 = jnp.bfloat16


def _cdiv(a, b):
    return -(-a // b)


def _placement(kw, w_in, pw, parity):
    j = np.arange(kw)[:, None, None]
    x = np.arange(w_in)[None, :, None]
    c = np.arange(pw)[None, None, :]
    return (x == 2 * c + parity + j).astype(np.float32)


_R1 = [_placement(5, 32, 14, p) for p in (0, 1)]
_R2 = [_placement(5, 14, 5, p) for p in (0, 1)]

_T1 = (np.arange(8)[None, :, None]
       == np.arange(4)[:, None, None] + np.arange(5)[None, None, :]
       ).astype(np.float32)

_T2 = (2 * np.arange(3)[None, :, None, None] + np.arange(2)[None, None, :, None]
       - np.arange(2)[:, None, None, None] == np.arange(5)[None, None, None, :]
       ).astype(np.float32)


def _fused_kernel(x_ref, w1_ref, c1b_ref, w2_ref, c2b_ref, w3_ref, b3_ref,
                  w4_ref, b4_ref, w5_ref, b5_ref, o_ref):
    tb = x_ref.shape[0]
    x8 = x_ref[...]

    lhs = jnp.concatenate([x8, jnp.roll(x8, -1, axis=1)], axis=-1)
    acc = jnp.dot(lhs.reshape(tb * 8, 256), w1_ref[...],
                  preferred_element_type=_F32).reshape(tb, 8, 1024)
    acc = jnp.maximum(acc[:, :, 0:512], acc[:, :, 512:1024])
    acc = jnp.maximum(acc[:, :, 0:256], acc[:, :, 256:512])
    a1 = jnp.maximum(acc + c1b_ref[...], 0.0).astype(_BF16)

    lhs = jnp.concatenate(
        [a1, jnp.roll(a1, -1, axis=1), jnp.roll(a1, -2, axis=1)], axis=-1)
    acc = jnp.dot(lhs.reshape(tb * 8, 768), w2_ref[...],
                  preferred_element_type=_F32).reshape(tb, 8, 512)
    acc = jnp.maximum(acc[:, :, 0:256], acc[:, :, 256:512])
    acc = jnp.maximum(acc[:, :, 0:128], acc[:, :, 128:256])
    a2 = jnp.maximum(acc + c2b_ref[...], 0.0).astype(_BF16)

    h = None
    for i in range(5):
        part = jnp.dot(a2[:, i, :], w3_ref[i * 128:(i + 1) * 128, :],
                       preferred_element_type=_F32)
        h = part if h is None else h + part
    h = jnp.maximum(h + b3_ref[...], 0.0).astype(_BF16)
    h = jnp.dot(h, w4_ref[...], preferred_element_type=_F32)
    h = jnp.maximum(h + b4_ref[...], 0.0).astype(_BF16)
    h = jnp.dot(h, w5_ref[...], preferred_element_type=_F32)
    o_ref[...] = (h + b5_ref[...]).astype(o_ref.dtype)


def kernel(x, w1, b1, w2, b2, w3, b3, w4, b4, w5, b5):
    B = x.shape[0]
    xs = x.reshape(B, 8, 128).astype(_BF16)

    w1t = jnp.transpose(w1.reshape(6, 5, 5), (1, 2, 0)).astype(_F32)
    w1_par = []
    for p in (0, 1):
        m = jnp.einsum('qdi,jwc,ijo->dwqco', jnp.asarray(_T1),
                       jnp.asarray(_R1[p]), w1t)
        m = m.reshape(8, 32, 2, 2, 84).transpose(0, 1, 3, 2, 4)
        w1_par.append(jnp.pad(m, ((0, 0),) * 4 + ((0, 44),)))
    w1m = jnp.stack(w1_par, axis=2).reshape(256, 1024).astype(_BF16)

    w2t = jnp.transpose(w2, (2, 3, 1, 0)).astype(_F32)
    w2_par = []
    for p in (0, 1):
        m = jnp.einsum('qgai,jrc,ijno->garnqco', jnp.asarray(_T2),
                       jnp.asarray(_R2[p]), w2t)
        w2_par.append(jnp.pad(m.reshape(3, 2, 84, 2, 80),
                              ((0, 0), (0, 0), (0, 44), (0, 0), (0, 48))))
    w2m = jnp.stack(w2_par, axis=3).reshape(768, 512).astype(_BF16)

    c1b = jnp.tile(jnp.pad(jnp.tile(b1.astype(_F32), 14), (0, 44)),
                   2).reshape(1, 256)
    c2b = jnp.pad(jnp.tile(b2.astype(_F32), 5), (0, 48)).reshape(1, 128)

    w3t = jnp.transpose(w3, (2, 3, 1, 0)).reshape(5, 80, 120).astype(_F32)
    w3m = jnp.pad(w3t, ((0, 0), (0, 48), (0, 0))).reshape(640, 120).astype(_BF16)
    w4t = w4.T.astype(_BF16)
    w5t = w5.T.astype(_BF16)
    b3r = b3.reshape(1, 120).astype(_F32)
    b4r = b4.reshape(1, 84).astype(_F32)
    b5r = b5.reshape(1, 10).astype(_F32)

    tb = 1024
    nb = _cdiv(B, tb)
    b_pad = nb * tb
    if b_pad != B:
        xs = jnp.pad(xs, ((0, b_pad - B), (0, 0), (0, 0)))

    out = pl.pallas_call(
        _fused_kernel,
        out_shape=jax.ShapeDtypeStruct((b_pad, 10), _F32),
        grid_spec=pltpu.PrefetchScalarGridSpec(
            num_scalar_prefetch=0,
            grid=(nb,),
            in_specs=[
                pl.BlockSpec((tb, 8, 128), lambda m: (m, 0, 0)),
                pl.BlockSpec((256, 1024), lambda m: (0, 0)),
                pl.BlockSpec((1, 256), lambda m: (0, 0)),
                pl.BlockSpec((768, 512), lambda m: (0, 0)),
                pl.BlockSpec((1, 128), lambda m: (0, 0)),
                pl.BlockSpec((640, 120), lambda m: (0, 0)),
                pl.BlockSpec((1, 120), lambda m: (0, 0)),
                pl.BlockSpec((120, 84), lambda m: (0, 0)),
                pl.BlockSpec((1, 84), lambda m: (0, 0)),
                pl.BlockSpec((84, 10), lambda m: (0, 0)),
                pl.BlockSpec((1, 10), lambda m: (0, 0)),
            ],
            out_specs=pl.BlockSpec((tb, 10), lambda m: (m, 0)),
        ),
        compiler_params=pltpu.CompilerParams(
            dimension_semantics=("parallel",),
            vmem_limit_bytes=64 * 1024 * 1024,
        ),
        cost_estimate=pl.CostEstimate(
            flops=2 * b_pad * (8 * 256 * 1024 + 8 * 768 * 512
                               + 5 * 128 * 128 + 128 * 128 + 128 * 128),
            transcendentals=0,
            bytes_accessed=4 * (b_pad * 32 * 32 + b_pad * 10),
        ),
    )(xs, w1m, c1b, w2m, c2b, w3m, b3r, w4t, b4r, w5t, b5r)
    return out[:B]
